# baseline (speedup 1.0000x reference)
import math
import functools

import jax
import jax.numpy as jnp
from jax.experimental import pallas as pl
from jax.experimental.pallas import tpu as pltpu


def _xh_body(x_ref, g_ref, b_ref, w1_ref, b1_ref, w2_ref, b2_ref, o_ref):
    x = x_ref[...]
    mu = jnp.mean(x, axis=1, keepdims=True)
    var = jnp.mean((x - mu) ** 2, axis=1, keepdims=True)
    xln = (x - mu) * jax.lax.rsqrt(var + 1e-5) * g_ref[...] + b_ref[...]
    t = jnp.dot(xln, w1_ref[...], preferred_element_type=jnp.float32) + b1_ref[...]
    s = t * jax.nn.sigmoid(t) * (1.0 / 0.6)
    o_ref[...] = jnp.dot(s, w2_ref[...], preferred_element_type=jnp.float32) + b2_ref[...]


def _rbf_body(r_ref, wr_ref, br_ref, o_ref):
    o_ref[...] = (
        jnp.dot(r_ref[...], wr_ref[...], preferred_element_type=jnp.float32)
        + br_ref[...]
    )


def _node_mlp(x, ln_gamma, ln_beta, W1, b1, W2, b2):
    n, h = x.shape
    bn = 1000
    grid = (n // bn,)
    full = lambda shape: pl.BlockSpec(shape, lambda i: (0,) * len(shape))
    return pl.pallas_call(
        _xh_body,
        grid=grid,
        in_specs=[
            pl.BlockSpec((bn, h), lambda i: (i, 0)),
            full((1, h)), full((1, h)),
            full((h, h)), full((1, h)),
            full((h, 3 * h)), full((1, 3 * h)),
        ],
        out_specs=pl.BlockSpec((bn, 3 * h), lambda i: (i, 0)),
        out_shape=jax.ShapeDtypeStruct((n, 3 * h), jnp.float32),
    )(x, ln_gamma.reshape(1, h), ln_beta.reshape(1, h),
      W1, b1.reshape(1, h), W2, b2.reshape(1, 3 * h))


def _rbf_mlp(edge_rbf, Wr, br):
    e, r = edge_rbf.shape
    c = Wr.shape[1]
    be = 4000
    grid = (e // be,)
    return pl.pallas_call(
        _rbf_body,
        grid=grid,
        in_specs=[
            pl.BlockSpec((be, r), lambda i: (i, 0)),
            pl.BlockSpec((r, c), lambda i: (0, 0)),
            pl.BlockSpec((1, c), lambda i: (0, 0)),
        ],
        out_specs=pl.BlockSpec((be, c), lambda i: (i, 0)),
        out_shape=jax.ShapeDtypeStruct((e, c), jnp.float32),
    )(edge_rbf, Wr, br.reshape(1, c))


def kernel(x, vec, edge_index, edge_rbf, edge_vector,
           ln_gamma, ln_beta, W1, b1, W2, b2, Wr, br):
    n, h = x.shape
    xh = _node_mlp(x, ln_gamma, ln_beta, W1, b1, W2, b2)
    rbfh = _rbf_mlp(edge_rbf, Wr, br)
    src = edge_index[0]
    dst = edge_index[1]
    xh_j = jnp.take(xh, src, axis=0)
    vec_j = jnp.take(vec, src, axis=0)
    m = xh_j * rbfh
    xm = m[:, :h]
    xh2 = m[:, h:2 * h] * (1.0 / math.sqrt(3.0))
    xh3 = m[:, 2 * h:]
    vecm = vec_j * xh2[:, None, :] + xh3[:, None, :] * edge_vector[:, :, None]
    vecm = vecm * (1.0 / math.sqrt(h))
    dx = jax.ops.segment_sum(xm, dst, num_segments=n)
    dvec = jax.ops.segment_sum(vecm, dst, num_segments=n)
    return (dx, dvec)


# speedup vs baseline: 8.6923x; 8.6923x over previous
import math
import functools

import jax
import jax.numpy as jnp
from jax import lax
from jax.experimental import pallas as pl
from jax.experimental.pallas import tpu as pltpu
from jax.experimental.pallas import tpu_sc as plsc

N = 10000
E = 320000
H = 128
G = 4
HG = H // G
ROW = 3 * HG
NC = 2
NS = 16
EPT = E // NS
K = 80
NBLK = EPT // K
NPT = 624
NTAIL = N - NS * NPT
CW = 256


def _xh_body(x_ref, vg_ref, g_ref, b_ref, w1_ref, b1_ref, w2_ref, b2_ref, o_ref):
    x = x_ref[...]
    mu = jnp.mean(x, axis=1, keepdims=True)
    var = jnp.mean((x - mu) ** 2, axis=1, keepdims=True)
    xln = (x - mu) * lax.rsqrt(var + 1e-5) * g_ref[...] + b_ref[...]
    t = jnp.dot(xln, w1_ref[...], preferred_element_type=jnp.float32) + b1_ref[...]
    s = t * jax.nn.sigmoid(t) * (1.0 / 0.6)
    xh = jnp.dot(s, w2_ref[0], preferred_element_type=jnp.float32) + b2_ref[0]
    pad = jnp.zeros((x.shape[0], CW - 2 * ROW), jnp.float32)
    o_ref[0] = jnp.concatenate([xh, vg_ref[0], pad], axis=1)


def _node_table_grouped(x, vec_g, ln_gamma, ln_beta, W1, b1, W2p, b2p):
    bn = 1000
    return pl.pallas_call(
        _xh_body,
        grid=(G, N // bn),
        in_specs=[
            pl.BlockSpec((bn, H), lambda g, i: (i, 0)),
            pl.BlockSpec((1, bn, ROW), lambda g, i: (g, i, 0)),
            pl.BlockSpec((1, H), lambda g, i: (0, 0)),
            pl.BlockSpec((1, H), lambda g, i: (0, 0)),
            pl.BlockSpec((H, H), lambda g, i: (0, 0)),
            pl.BlockSpec((1, H), lambda g, i: (0, 0)),
            pl.BlockSpec((1, H, ROW), lambda g, i: (g, 0, 0)),
            pl.BlockSpec((1, 1, ROW), lambda g, i: (g, 0, 0)),
        ],
        out_specs=pl.BlockSpec((1, bn, CW), lambda g, i: (g, i, 0)),
        out_shape=jax.ShapeDtypeStruct((G, N, CW), jnp.float32),
    )(x, vec_g, ln_gamma.reshape(1, H), ln_beta.reshape(1, H),
      W1, b1.reshape(1, H), W2p, b2p)


def _rbf_body(r_ref, wr_ref, br_ref, o_ref):
    y = (jnp.dot(r_ref[...], wr_ref[...], preferred_element_type=jnp.float32)
         + br_ref[...])
    for g in range(G):
        o_ref[g] = y[:, g * ROW:(g + 1) * ROW]


def _rbf_mlp_grouped(edge_rbf, Wrp, brp):
    r = edge_rbf.shape[1]
    be = 2000
    return pl.pallas_call(
        _rbf_body,
        grid=(E // be,),
        in_specs=[
            pl.BlockSpec((be, r), lambda i: (i, 0)),
            pl.BlockSpec((r, G * ROW), lambda i: (0, 0)),
            pl.BlockSpec((1, G * ROW), lambda i: (0, 0)),
        ],
        out_specs=pl.BlockSpec((G, be, ROW), lambda i: (0, i, 0)),
        out_shape=jax.ShapeDtypeStruct((G, E, ROW), jnp.float32),
    )(edge_rbf, Wrp, brp.reshape(1, G * ROW))


def _sc_body(comb, rbfg, src_h, dst_h, ev_h, zeros_h, out_h,
             srcb, dstb, cbuf, rbb, evb, outb, acc,
             sem_a, sem_b):
    c = lax.axis_index("c")
    t = lax.axis_index("s")

    for gi in range(2):
        g = 2 * c + gi

        pltpu.sync_copy(zeros_h.at[pl.ds(t * NPT, NPT)],
                        acc.at[pl.ds(t * NPT, NPT)])

        @pl.when(t == 0)
        def _zero_tail():
            pltpu.sync_copy(zeros_h.at[pl.ds(NS * NPT, NTAIL)],
                            acc.at[pl.ds(NS * NPT, NTAIL)])

        plsc.subcore_barrier()

        @pl.loop(0, NBLK)
        def _block(b):
            e0 = t * EPT + b * K
            pltpu.sync_copy(src_h.at[pl.ds(e0, K)], srcb)
            pltpu.sync_copy(dst_h.at[pl.ds(e0, K)], dstb)

            @pl.loop(0, K // 16)
            def _adj(i):
                srcb[pl.ds(i * 16, 16)] = srcb[pl.ds(i * 16, 16)] + g * N

            ga = pltpu.async_copy(comb.at[srcb], cbuf, sem_a)
            pltpu.sync_copy(rbfg.at[pl.ds(g * E + e0, K)], rbb)
            pltpu.sync_copy(ev_h.at[pl.ds(3 * e0, 3 * K)], evb.at[pl.ds(0, 3 * K)])
            ga.wait()

            @pl.loop(0, K, unroll=2)
            def _edge(j):
                ev = evb[pl.ds(3 * j, 16)]
                ev0 = ev[0]
                ev1 = ev[1]
                ev2 = ev[2]
                for k in range(HG // 16):
                    o = k * 16
                    xh1 = cbuf[j, pl.ds(o, 16)]
                    rb1 = rbb[j, pl.ds(o, 16)]
                    outb[j, pl.ds(o, 16)] = xh1 * rb1
                    m2 = cbuf[j, pl.ds(HG + o, 16)] * rbb[j, pl.ds(HG + o, 16)]
                    m3 = cbuf[j, pl.ds(2 * HG + o, 16)] * rbb[j, pl.ds(2 * HG + o, 16)]
                    outb[j, pl.ds(HG + o, 16)] = cbuf[j, pl.ds(ROW + o, 16)] * m2 + m3 * ev0
                    outb[j, pl.ds(2 * HG + o, 16)] = cbuf[j, pl.ds(ROW + HG + o, 16)] * m2 + m3 * ev1
                    outb[j, pl.ds(3 * HG + o, 16)] = cbuf[j, pl.ds(ROW + 2 * HG + o, 16)] * m2 + m3 * ev2

            pltpu.sync_copy(outb, acc.at[dstb], add=True)

        plsc.subcore_barrier()

        pltpu.sync_copy(acc.at[pl.ds(t * NPT, NPT)],
                        out_h.at[pl.ds(g * N + t * NPT, NPT)])

        @pl.when(t == 0)
        def _flush_tail():
            pltpu.sync_copy(acc.at[pl.ds(NS * NPT, NTAIL)],
                            out_h.at[pl.ds(g * N + NS * NPT, NTAIL)])


_sc_aggregate = functools.partial(
    pl.kernel,
    _sc_body,
    out_type=jax.ShapeDtypeStruct((G * N, H), jnp.float32),
    mesh=plsc.VectorSubcoreMesh(core_axis_name="c", subcore_axis_name="s"),
    scratch_types=[
        pltpu.VMEM((K,), jnp.int32),
        pltpu.VMEM((K,), jnp.int32),
        pltpu.VMEM((K, CW), jnp.float32),
        pltpu.VMEM((K, ROW), jnp.float32),
        pltpu.VMEM((3 * K + 16,), jnp.float32),
        pltpu.VMEM((K, H), jnp.float32),
        pltpu.VMEM_SHARED((N, H), jnp.float32),
        pltpu.SemaphoreType.DMA,
        pltpu.SemaphoreType.DMA,
    ],
)()


def kernel(x, vec, edge_index, edge_rbf, edge_vector,
           ln_gamma, ln_beta, W1, b1, W2, b2, Wr, br):
    perm = []
    for g in range(G):
        for part in range(3):
            perm.extend(range(part * H + g * HG, part * H + (g + 1) * HG))
    perm = jnp.asarray(perm, dtype=jnp.int32)
    scale = jnp.concatenate([
        jnp.full((H,), 1.0, jnp.float32),
        jnp.full((H,), 1.0 / (math.sqrt(3.0) * math.sqrt(H)), jnp.float32),
        jnp.full((H,), 1.0 / math.sqrt(H), jnp.float32),
    ])
    W2p = W2[:, perm].reshape(H, G, ROW).transpose(1, 0, 2)
    b2p = b2[perm].reshape(G, 1, ROW)
    Wrp = (Wr * scale[None, :])[:, perm]
    brp = (br * scale)[perm]

    vec_g = vec.reshape(N, 3, G, HG).transpose(2, 0, 1, 3).reshape(G, N, ROW)
    comb = _node_table_grouped(x, vec_g, ln_gamma, ln_beta, W1, b1, W2p, b2p)
    rbf_g = _rbf_mlp_grouped(edge_rbf, Wrp, brp)

    src = edge_index[0]
    dst = edge_index[1]
    ev_flat = edge_vector.reshape(3 * E)

    acc = _sc_aggregate(
        comb.reshape(G * N, CW), rbf_g.reshape(G * E, ROW),
        src, dst, ev_flat, jnp.zeros((N, H), jnp.float32),
    )

    acc = acc.reshape(G, N, H)
    dx = acc[:, :, :HG].transpose(1, 0, 2).reshape(N, H)
    dvec = (acc[:, :, HG:].reshape(G, N, 3, HG)
            .transpose(1, 2, 0, 3).reshape(N, 3, H))
    return (dx, dvec)
